# baseline (speedup 1.0000x reference)
import functools

import jax
import jax.numpy as jnp
from jax import lax
from jax.experimental import pallas as pl
from jax.experimental.pallas import tpu as pltpu
from jax.experimental.pallas import tpu_sc as plsc

U = 1024
DIM = 64
DEPTH = 16

BATCH = 4096
SEQ = 200
NC, NS = 2, 16
NW = NC * NS
LW = 128
NBG = LW // 16
NPAIR = SEQ // 2


def _embed_body(unique_ref, prim_ref, out_ref):
    u = unique_ref[:]
    m0 = prim_ref[0]
    m1 = prim_ref[1]
    mapsT = jnp.ones((DIM, U), jnp.float32)
    dn = (((1,), (0,)), ((), ()))
    for depth in range(DEPTH):
        shifted = u >> depth
        code = jnp.where(shifted > 1, shifted & 1, 2)
        a = lax.dot_general(m0, mapsT, dn, preferred_element_type=jnp.float32)
        b = lax.dot_general(m1, mapsT, dn, preferred_element_type=jnp.float32)
        mapsT = jnp.where(code == 0, a, jnp.where(code == 1, b, mapsT))
    out_ref[:] = mapsT


def _embed(unique, primitives):
    return pl.pallas_call(
        _embed_body,
        out_shape=jax.ShapeDtypeStruct((DIM, U), jnp.float32),
    )(unique.reshape(1, U), primitives)


def _gather_body(mapT_hbm, tableT_hbm, out_hbm,
                 table_v, idx_v, st0, st1, os0, os1):
    wid = lax.axis_index("s") * NC + lax.axis_index("c")
    lane0 = wid * LW
    sts = (st0, st1)
    oss = (os0, os1)

    pltpu.sync_copy(tableT_hbm, table_v)
    pltpu.sync_copy(mapT_hbm.at[:, pl.ds(lane0, LW)], idx_v)

    LAT = 6
    NH = 4
    HD = DIM // NH

    def compute_half(st, idxs, h):
        items = [(d, bg) for d in range(h * HD, h * HD + HD)
                 for bg in range(NBG)]
        vals = {}
        for i in range(len(items) + LAT):
            if i < len(items):
                d, bg = items[i]
                vals[i] = plsc.load_gather(
                    table_v.at[pl.ds(d * U, U)], [idxs[bg]])
            if i >= LAT:
                d, bg = items[i - LAT]
                st[0, d, pl.ds(bg * 16, 16)] = vals.pop(i - LAT)

    def fire_half(st, osem, s, h):
        pltpu.async_copy(
            st.at[:, pl.ds(h * HD, HD), :],
            out_hbm.at[pl.ds(s, 1), pl.ds(h * HD, HD), pl.ds(lane0, LW)],
            osem)

    def wait_half(st, osem, h):
        pltpu.make_async_copy(
            st.at[:, pl.ds(h * HD, HD), :],
            out_hbm.at[pl.ds(0, 1), pl.ds(h * HD, HD), pl.ds(lane0, LW)],
            osem).wait()

    def pair(p, _):
        s0 = 2 * p
        for k in range(2):
            s = s0 + k
            idxs = [idx_v[s, pl.ds(bg * 16, 16)] for bg in range(NBG)]
            for h in range(NH):
                @pl.when(p > 0)
                def _(k=k, h=h):
                    wait_half(sts[k], oss[k], h)
                compute_half(sts[k], idxs, h)
                fire_half(sts[k], oss[k], s, h)
        return ()

    lax.fori_loop(0, NPAIR, pair, (), unroll=False)
    for k in range(2):
        for h in range(NH):
            wait_half(sts[k], oss[k], h)


@functools.partial(jax.jit, static_argnums=())
def _gather(mapT, tableT_flat):
    mesh = plsc.VectorSubcoreMesh(core_axis_name="c", subcore_axis_name="s")
    f = pl.kernel(
        _gather_body,
        out_type=jax.ShapeDtypeStruct((SEQ, DIM, BATCH), jnp.float32),
        mesh=mesh,
        scratch_types=[
            pltpu.VMEM((DIM * U,), jnp.float32),
            pltpu.VMEM((SEQ, LW), jnp.int32),
            pltpu.VMEM((1, DIM, LW), jnp.float32),
            pltpu.VMEM((1, DIM, LW), jnp.float32),
            pltpu.SemaphoreType.DMA,
            pltpu.SemaphoreType.DMA,
        ],
        compiler_params=pltpu.CompilerParams(
            use_tc_tiling_on_sc=True, needs_layout_passes=False),
    )
    return f(mapT, tableT_flat)


def kernel(unique, mapping, primitives):
    tableT = _embed(unique, primitives)
    outP = _gather(mapping.T, tableT.reshape(DIM * U))
    return jnp.transpose(outP, (2, 0, 1))
